# Initial kernel scaffold; baseline (speedup 1.0000x reference)
import functools

import jax
import jax.numpy as jnp
from jax import lax
from jax.experimental import pallas as pl
from jax.experimental.pallas import tpu as pltpu
from jax.experimental.pallas import tpu_sc as plsc

NC = 2
NS = 16
NW = NC * NS
LANES = 16

D = 64
DM = D + 1
CHUNK = 160
EPS = 1e-15


def _rsqrt(s):
    i = plsc.bitcast(s, jnp.int32)
    i = jnp.int32(0x5F3759DF) - lax.shift_right_arithmetic(i, 1)
    r = plsc.bitcast(i, jnp.float32)
    for _ in range(3):
        r = r * (1.5 - 0.5 * s * r * r)
    return r


def _softplus(c):
    e = jnp.exp(-jnp.abs(c))
    z = e / (e + 2.0)
    z2 = z * z
    p = jnp.float32(1.0 / 9.0)
    p = 1.0 / 7.0 + z2 * p
    p = 1.0 / 5.0 + z2 * p
    p = 1.0 / 3.0 + z2 * p
    p = 1.0 + z2 * p
    return jnp.maximum(c, 0.0) + 2.0 * z * p


def _sc_body(n_lookups, x_hbm, mean_hbm, covar_hbm, outm_hbm, outc_hbm,
             idx_v, mrows, crows, outm_v, outc_v, scale_v, sem_m, sem_c):
    per_w = n_lookups // NW
    n_chunks = per_w // CHUNK
    wid = lax.axis_index("s") * NC + lax.axis_index("c")
    base = wid * per_w
    iota = lax.iota(jnp.int32, LANES)

    def chunk_body(ci, carry):
        off = base + ci * CHUNK
        pltpu.sync_copy(x_hbm.at[pl.ds(off, CHUNK)], idx_v)
        cm = pltpu.async_copy(mean_hbm.at[idx_v], mrows, sem_m)
        cc = pltpu.async_copy(covar_hbm.at[idx_v], crows, sem_c)
        cm.wait()
        cc.wait()

        def grp(gi, c2):
            row0 = gi * LANES

            def dloop(d, acc):
                v = plsc.load_gather(
                    mrows, [row0 + iota, jnp.full((LANES,), d, jnp.int32)])
                return acc + v * v

            ssum = lax.fori_loop(0, D, dloop, jnp.zeros((LANES,), jnp.float32))
            s = jnp.maximum(ssum, EPS)
            r = _rsqrt(s)
            n = s * r
            e = jnp.exp(n)
            ei = 1.0 / e
            cosh = 0.5 * (e + ei)
            scale = (0.5 * (e - ei)) * r
            scale_v[pl.ds(row0, LANES)] = scale
            plsc.store_scatter(outm_v, [(row0 + iota) * DM], cosh)
            return c2

        lax.fori_loop(0, CHUNK // LANES, grp, 0)

        def sc2(k, c2):
            l = k // 4
            col = (k % 4) * LANES
            splat = plsc.load_gather(scale_v, [jnp.full((LANES,), l, jnp.int32)])
            m = mrows[l, pl.ds(col, LANES)]
            plsc.store_scatter(outm_v, [l * DM + 1 + col + iota], m * splat)
            return c2

        lax.fori_loop(0, CHUNK * 4, sc2, 0)

        def sp3(k, c2):
            l = k // 4
            col = (k % 4) * LANES
            cv = crows[l, pl.ds(col, LANES)]
            outc_v[pl.ds(k * LANES, LANES)] = _softplus(cv)
            return c2

        lax.fori_loop(0, CHUNK * 4, sp3, 0)

        pltpu.sync_copy(outm_v, outm_hbm.at[pl.ds(off * DM, CHUNK * DM)])
        pltpu.sync_copy(outc_v, outc_hbm.at[pl.ds(off * D, CHUNK * D)])
        return carry

    lax.fori_loop(0, n_chunks, chunk_body, 0)


@functools.partial(jax.jit, static_argnames=("n_lookups",))
def _run(x_flat, mean_table, covar_table, n_lookups):
    mesh = plsc.VectorSubcoreMesh(
        core_axis_name="c", subcore_axis_name="s",
        num_cores=NC, num_subcores=NS)
    fn = pl.kernel(
        functools.partial(_sc_body, n_lookups),
        out_type=(
            jax.ShapeDtypeStruct((n_lookups * DM,), jnp.float32),
            jax.ShapeDtypeStruct((n_lookups * D,), jnp.float32),
        ),
        mesh=mesh,
        scratch_types=[
            pltpu.VMEM((CHUNK,), jnp.int32),
            pltpu.VMEM((CHUNK, D), jnp.float32),
            pltpu.VMEM((CHUNK, D), jnp.float32),
            pltpu.VMEM((CHUNK * DM,), jnp.float32),
            pltpu.VMEM((CHUNK * D,), jnp.float32),
            pltpu.VMEM((CHUNK,), jnp.float32),
            pltpu.SemaphoreType.DMA,
            pltpu.SemaphoreType.DMA,
        ],
    )
    return fn(x_flat, mean_table, covar_table)


def kernel(x, mean_table, covar_table):
    b, l = x.shape
    n = b * l
    outm, outc = _run(x.reshape(n), mean_table, covar_table, n)
    return outm.reshape(b, l, DM), outc.reshape(b, l, D)


# speedup vs baseline: 1.5619x; 1.5619x over previous
import functools

import jax
import jax.numpy as jnp
from jax import lax
from jax.experimental import pallas as pl
from jax.experimental.pallas import tpu as pltpu
from jax.experimental.pallas import tpu_sc as plsc

NC = 2
NS = 16
NW = NC * NS
LANES = 16

D = 64
DM = D + 1
CHUNK = 160
EPS = 1e-15


def _rsqrt(s):
    i = plsc.bitcast(s, jnp.int32)
    i = jnp.int32(0x5F3759DF) - lax.shift_right_arithmetic(i, 1)
    r = plsc.bitcast(i, jnp.float32)
    for _ in range(3):
        r = r * (1.5 - 0.5 * s * r * r)
    return r


def _softplus(c):
    e = jnp.exp(-jnp.abs(c))
    z = e / (e + 2.0)
    z2 = z * z
    p = jnp.float32(1.0 / 9.0)
    p = 1.0 / 7.0 + z2 * p
    p = 1.0 / 5.0 + z2 * p
    p = 1.0 / 3.0 + z2 * p
    p = 1.0 + z2 * p
    return jnp.maximum(c, 0.0) + 2.0 * z * p


def _sc_body(n_lookups, x_hbm, mean_hbm, covar_hbm, outm_hbm, outc_hbm,
             idx_v, mrows, crows, outm_v, outc_v, scale_v, tmp_v, sem_m, sem_c):
    per_w = n_lookups // NW
    n_chunks = per_w // CHUNK
    wid = lax.axis_index("s") * NC + lax.axis_index("c")
    base = wid * per_w
    iota = lax.iota(jnp.int32, LANES)

    def chunk_body(ci, carry):
        off = base + ci * CHUNK
        pltpu.sync_copy(x_hbm.at[pl.ds(off, CHUNK)], idx_v)
        cm = pltpu.async_copy(mean_hbm.at[idx_v], mrows, sem_m)
        cc = pltpu.async_copy(covar_hbm.at[idx_v], crows, sem_c)
        cm.wait()
        cc.wait()

        def grp(gi, c2):
            row0 = gi * LANES

            def ll(l, c3):
                acc = jnp.zeros((LANES,), jnp.float32)
                for j in range(4):
                    m = mrows[row0 + l, pl.ds(j * LANES, LANES)]
                    acc = acc + m * m
                plsc.store_scatter(tmp_v, [iota * LANES + l], acc)
                return c3

            lax.fori_loop(0, LANES, ll, 0)

            def rr(t, acc):
                return acc + tmp_v[pl.ds(t * LANES, LANES)]

            ssum = lax.fori_loop(0, LANES, rr, jnp.zeros((LANES,), jnp.float32))
            s = jnp.maximum(ssum, EPS)
            r = _rsqrt(s)
            n = s * r
            e = jnp.exp(n)
            ei = 1.0 / e
            cosh = 0.5 * (e + ei)
            scale = (0.5 * (e - ei)) * r
            scale_v[pl.ds(row0, LANES)] = scale
            plsc.store_scatter(outm_v, [(row0 + iota) * DM], cosh)
            return c2

        lax.fori_loop(0, CHUNK // LANES, grp, 0)

        def sc2(k, c2):
            l = k // 4
            col = (k % 4) * LANES
            splat = plsc.load_gather(scale_v, [jnp.full((LANES,), l, jnp.int32)])
            m = mrows[l, pl.ds(col, LANES)]
            plsc.store_scatter(outm_v, [l * DM + 1 + col + iota], m * splat)
            return c2

        lax.fori_loop(0, CHUNK * 4, sc2, 0)

        def sp3(k, c2):
            l = k // 4
            col = (k % 4) * LANES
            cv = crows[l, pl.ds(col, LANES)]
            outc_v[pl.ds(k * LANES, LANES)] = _softplus(cv)
            return c2

        lax.fori_loop(0, CHUNK * 4, sp3, 0)

        pltpu.sync_copy(outm_v, outm_hbm.at[pl.ds(off * DM, CHUNK * DM)])
        pltpu.sync_copy(outc_v, outc_hbm.at[pl.ds(off * D, CHUNK * D)])
        return carry

    lax.fori_loop(0, n_chunks, chunk_body, 0)


@functools.partial(jax.jit, static_argnames=("n_lookups",))
def _run(x_flat, mean_table, covar_table, n_lookups):
    mesh = plsc.VectorSubcoreMesh(
        core_axis_name="c", subcore_axis_name="s",
        num_cores=NC, num_subcores=NS)
    fn = pl.kernel(
        functools.partial(_sc_body, n_lookups),
        out_type=(
            jax.ShapeDtypeStruct((n_lookups * DM,), jnp.float32),
            jax.ShapeDtypeStruct((n_lookups * D,), jnp.float32),
        ),
        mesh=mesh,
        compiler_params=pltpu.CompilerParams(
            needs_layout_passes=False, use_tc_tiling_on_sc=False),
        scratch_types=[
            pltpu.VMEM((CHUNK,), jnp.int32),
            pltpu.VMEM((CHUNK, D), jnp.float32),
            pltpu.VMEM((CHUNK, D), jnp.float32),
            pltpu.VMEM((CHUNK * DM,), jnp.float32),
            pltpu.VMEM((CHUNK * D,), jnp.float32),
            pltpu.VMEM((CHUNK,), jnp.float32),
            pltpu.VMEM((LANES * LANES,), jnp.float32),
            pltpu.SemaphoreType.DMA,
            pltpu.SemaphoreType.DMA,
        ],
    )
    return fn(x_flat, mean_table, covar_table)


def kernel(x, mean_table, covar_table):
    b, l = x.shape
    n = b * l
    outm, outc = _run(x.reshape(n), mean_table, covar_table, n)
    return outm.reshape(b, l, DM), outc.reshape(b, l, D)
